# Initial kernel scaffold; baseline (speedup 1.0000x reference)
import jax
import jax.numpy as jnp
from jax.experimental import pallas as pl


def kernel(token_ids, weight):
    raise NotImplementedError("write your pallas kernel here")


# speedup vs baseline: 1.1127x; 1.1127x over previous
import functools

import jax
import jax.numpy as jnp
from jax import lax
from jax.experimental import pallas as pl
from jax.experimental.pallas import tpu as pltpu
from jax.experimental.pallas import tpu_sc as plsc

_NUM_EMBEDDINGS = 1000000
_D = 32
_B = 16384
_H = 50
_TOTAL = _B * _H
_NW = 32
_PER_W = _TOTAL // _NW
_CHUNK = 1024
_NCHUNK = _PER_W // _CHUNK
_GATHER = 128
_NGATHER = _CHUNK // _GATHER


def _emb_body(idx_hbm, table_hbm, out_hbm, idx_v, rows_v, gsem, wsem):
    wid = lax.axis_index("s") * 2 + lax.axis_index("c")
    base = wid * _PER_W

    def idx_src(g):
        row = pl.multiple_of((base + g * _CHUNK) // _GATHER, 8)
        return idx_hbm.at[pl.ds(row, _NGATHER)]

    def out_dst(g):
        return out_hbm.at[pl.ds(base + g * _CHUNK, _CHUNK)]

    def fire_gathers(slot):
        for j in range(_NGATHER):
            pltpu.async_copy(
                table_hbm.at[idx_v.at[slot, j]],
                rows_v.at[slot, pl.ds(j * _GATHER, _GATHER)],
                gsem.at[slot],
            )

    def drain_gathers(slot):
        pltpu.make_async_copy(
            table_hbm.at[pl.ds(0, _CHUNK)], rows_v.at[slot], gsem.at[slot]
        ).wait()

    pltpu.sync_copy(idx_src(0), idx_v.at[0])
    fire_gathers(0)

    def body(g, _):
        slot = lax.rem(g, 2)
        nslot = 1 - slot

        @pl.when(g + 1 < _NCHUNK)
        def _prefetch():
            pltpu.sync_copy(idx_src(g + 1), idx_v.at[nslot])

            @pl.when(g >= 1)
            def _drain_write():
                pltpu.make_async_copy(
                    rows_v.at[nslot], out_dst(g - 1), wsem.at[nslot]
                ).wait()

            fire_gathers(nslot)

        drain_gathers(slot)
        pltpu.async_copy(rows_v.at[slot], out_dst(g), wsem.at[slot])
        return _

    lax.fori_loop(0, _NCHUNK, body, None)

    last = _NCHUNK - 1
    pltpu.make_async_copy(
        rows_v.at[lax.rem(last, 2)], out_dst(last), wsem.at[lax.rem(last, 2)]
    ).wait()

    @pl.when(_NCHUNK >= 2)
    def _():
        pltpu.make_async_copy(
            rows_v.at[lax.rem(last - 1, 2)],
            out_dst(last - 1),
            wsem.at[lax.rem(last - 1, 2)],
        ).wait()


@jax.jit
def _emb(token_ids_flat, weight):
    mesh = plsc.VectorSubcoreMesh(core_axis_name="c", subcore_axis_name="s")
    run = functools.partial(
        pl.kernel,
        mesh=mesh,
        out_type=jax.ShapeDtypeStruct((_TOTAL, _D), jnp.float32),
        scratch_types=[
            pltpu.VMEM((2, _NGATHER, _GATHER), jnp.int32),
            pltpu.VMEM((2, _CHUNK, _D), jnp.float32),
            pltpu.SemaphoreType.DMA((2,)),
            pltpu.SemaphoreType.DMA((2,)),
        ],
        compiler_params=pltpu.CompilerParams(use_tc_tiling_on_sc=False),
    )(_emb_body)
    return run(token_ids_flat, weight)


def kernel(token_ids, weight):
    flat = token_ids.reshape(_TOTAL // _GATHER, _GATHER).astype(jnp.int32)
    out = _emb(flat, weight)
    return out.reshape(token_ids.shape + (_D,))


# speedup vs baseline: 1.7862x; 1.6053x over previous
import functools

import jax
import jax.numpy as jnp
from jax import lax
from jax.experimental import pallas as pl
from jax.experimental.pallas import tpu as pltpu
from jax.experimental.pallas import tpu_sc as plsc

_NUM_EMBEDDINGS = 1000000
_D = 32
_B = 16384
_H = 50
_NW = 32
_ROWS_W = _B // _NW
_NB = 16
_NCHUNK = _ROWS_W // _NB


def _emb_body(idx_hbm, table_hbm, out_hbm, idx_v, rows_v, gsem, wsem):
    wid = lax.axis_index("s") * 2 + lax.axis_index("c")
    base = wid * _ROWS_W

    def fire_chunk(g, slot):
        pltpu.sync_copy(idx_hbm.at[pl.ds(base + g * _NB, _NB)], idx_v.at[slot])
        for i in range(_NB):
            pltpu.async_copy(
                table_hbm.at[idx_v.at[slot, i]],
                rows_v.at[slot, i],
                gsem.at[slot],
            )

    def drain_gathers(slot):
        pltpu.make_async_copy(
            out_hbm.at[pl.ds(0, _NB)], rows_v.at[slot], gsem.at[slot]
        ).wait()

    def out_dst(g):
        return out_hbm.at[pl.ds(base + g * _NB, _NB)]

    fire_chunk(0, 0)

    def body(g, _):
        slot = lax.rem(g, 2)
        nslot = 1 - slot

        @pl.when(g + 1 < _NCHUNK)
        def _prefetch():
            @pl.when(g >= 1)
            def _drain_write():
                pltpu.make_async_copy(
                    rows_v.at[nslot], out_dst(g - 1), wsem.at[nslot]
                ).wait()

            fire_chunk(g + 1, nslot)

        drain_gathers(slot)
        pltpu.async_copy(rows_v.at[slot], out_dst(g), wsem.at[slot])
        return _

    lax.fori_loop(0, _NCHUNK, body, None)

    last = _NCHUNK - 1
    pltpu.make_async_copy(
        rows_v.at[lax.rem(last, 2)], out_dst(last), wsem.at[lax.rem(last, 2)]
    ).wait()

    @pl.when(_NCHUNK >= 2)
    def _():
        pltpu.make_async_copy(
            rows_v.at[lax.rem(last - 1, 2)],
            out_dst(last - 1),
            wsem.at[lax.rem(last - 1, 2)],
        ).wait()


@jax.jit
def _emb(token_ids, weight):
    mesh = plsc.VectorSubcoreMesh(core_axis_name="c", subcore_axis_name="s")
    run = functools.partial(
        pl.kernel,
        mesh=mesh,
        out_type=jax.ShapeDtypeStruct((_B, _H, _D), jnp.float32),
        scratch_types=[
            pltpu.VMEM((2, _NB, _H), jnp.int32),
            pltpu.VMEM((2, _NB, _H, _D), jnp.float32),
            pltpu.SemaphoreType.DMA((2,)),
            pltpu.SemaphoreType.DMA((2,)),
        ],
        compiler_params=pltpu.CompilerParams(use_tc_tiling_on_sc=False),
    )(_emb_body)
    return run(token_ids, weight)


def kernel(token_ids, weight):
    return _emb(token_ids.astype(jnp.int32), weight)


# speedup vs baseline: 1.9204x; 1.0752x over previous
import functools

import jax
import jax.numpy as jnp
from jax import lax
from jax.experimental import pallas as pl
from jax.experimental.pallas import tpu as pltpu
from jax.experimental.pallas import tpu_sc as plsc

_NUM_EMBEDDINGS = 1000000
_D = 32
_B = 16384
_H = 50
_NW = 32
_GROUPS = (_B // 128) * _H
_GPW = _GROUPS // _NW
_GPC = 8
_CHUNK = _GPC * 128
_NCHUNK = _GPW // _GPC


def _emb_body(idx_hbm, table_hbm, out_hbm, idx_v, rows_v, gsem, wsem):
    wid = lax.axis_index("s") * 2 + lax.axis_index("c")
    g0 = wid * _GPW

    def chunk_pos(c):
        g = g0 + c * _GPC
        h = g // 128
        off = (g % 128) * 128
        return h, off

    def fire_chunk(c, slot):
        h, off = chunk_pos(c)
        pltpu.sync_copy(idx_hbm.at[h, pl.ds(off, _CHUNK)], idx_v.at[slot])
        for j in range(_GPC):
            pltpu.async_copy(
                table_hbm.at[idx_v.at[slot, pl.ds(j * 128, 128)]],
                rows_v.at[slot, pl.ds(j * 128, 128)],
                gsem.at[slot],
            )

    def drain_gathers(slot):
        pltpu.make_async_copy(
            out_hbm.at[0, pl.ds(0, _CHUNK)], rows_v.at[slot], gsem.at[slot]
        ).wait()

    def out_dst(c):
        h, off = chunk_pos(c)
        return out_hbm.at[h, pl.ds(off, _CHUNK)]

    fire_chunk(0, 0)

    def body(c, _):
        slot = lax.rem(c, 2)
        nslot = 1 - slot

        @pl.when(c + 1 < _NCHUNK)
        def _prefetch():
            @pl.when(c >= 1)
            def _drain_write():
                pltpu.make_async_copy(
                    rows_v.at[nslot], out_dst(c - 1), wsem.at[nslot]
                ).wait()

            fire_chunk(c + 1, nslot)

        drain_gathers(slot)
        pltpu.async_copy(rows_v.at[slot], out_dst(c), wsem.at[slot])
        return _

    lax.fori_loop(0, _NCHUNK, body, None)

    last = _NCHUNK - 1
    pltpu.make_async_copy(
        rows_v.at[lax.rem(last, 2)], out_dst(last), wsem.at[lax.rem(last, 2)]
    ).wait()

    @pl.when(_NCHUNK >= 2)
    def _():
        pltpu.make_async_copy(
            rows_v.at[lax.rem(last - 1, 2)],
            out_dst(last - 1),
            wsem.at[lax.rem(last - 1, 2)],
        ).wait()


@jax.jit
def _emb(token_ids_t, weight):
    mesh = plsc.VectorSubcoreMesh(core_axis_name="c", subcore_axis_name="s")
    run = functools.partial(
        pl.kernel,
        mesh=mesh,
        out_type=jax.ShapeDtypeStruct((_H, _B, _D), jnp.float32),
        scratch_types=[
            pltpu.VMEM((2, _CHUNK), jnp.int32),
            pltpu.VMEM((2, _CHUNK, _D), jnp.float32),
            pltpu.SemaphoreType.DMA((2,)),
            pltpu.SemaphoreType.DMA((2,)),
        ],
        compiler_params=pltpu.CompilerParams(use_tc_tiling_on_sc=False),
    )(_emb_body)
    return run(token_ids_t, weight)


def kernel(token_ids, weight):
    out_t = _emb(token_ids.astype(jnp.int32).T, weight)
    return out_t.transpose(1, 0, 2)


# speedup vs baseline: 1.9442x; 1.0124x over previous
import functools

import jax
import jax.numpy as jnp
from jax import lax
from jax.experimental import pallas as pl
from jax.experimental.pallas import tpu as pltpu
from jax.experimental.pallas import tpu_sc as plsc

_NUM_EMBEDDINGS = 1000000
_D = 32
_B = 16384
_H = 50
_NW = 32
_GROUPS = (_B // 128) * _H
_GPW = _GROUPS // _NW
_GPC = 8
_CHUNK = _GPC * 128
_NCHUNK = _GPW // _GPC


def _emb_body(idx_hbm, table_hbm, out_hbm, idx_v, rows_v, gsem, wsem):
    wid = lax.axis_index("s") * 2 + lax.axis_index("c")
    g0 = wid * _GPW

    def chunk_pos(c):
        g = g0 + c * _GPC
        h = g // 128
        off = (g % 128) * 128
        return h, off

    def fire_chunk(c, slot):
        h, off = chunk_pos(c)
        pltpu.sync_copy(idx_hbm.at[h, pl.ds(off, _CHUNK)], idx_v.at[slot])
        for j in range(_GPC):
            pltpu.async_copy(
                table_hbm.at[idx_v.at[slot, pl.ds(j * 128, 128)]],
                rows_v.at[slot, pl.ds(j * 128, 128)],
                gsem.at[slot],
            )

    def drain_gathers(slot):
        pltpu.make_async_copy(
            out_hbm.at[0, pl.ds(0, _CHUNK)], rows_v.at[slot], gsem.at[slot]
        ).wait()

    def out_dst(c):
        h, off = chunk_pos(c)
        return out_hbm.at[h, pl.ds(off, _CHUNK)]

    fire_chunk(0, 0)

    def body(c, _):
        slot = lax.rem(c, 2)
        nslot = 1 - slot

        @pl.when(c + 1 < _NCHUNK)
        def _prefetch():
            @pl.when(c >= 1)
            def _drain_write():
                pltpu.make_async_copy(
                    rows_v.at[nslot], out_dst(c - 1), wsem.at[nslot]
                ).wait()

            fire_chunk(c + 1, nslot)

        drain_gathers(slot)
        pltpu.async_copy(rows_v.at[slot], out_dst(c), wsem.at[slot])
        return _

    lax.fori_loop(0, _NCHUNK, body, None)

    last = _NCHUNK - 1
    pltpu.make_async_copy(
        rows_v.at[lax.rem(last, 2)], out_dst(last), wsem.at[lax.rem(last, 2)]
    ).wait()

    @pl.when(_NCHUNK >= 2)
    def _():
        pltpu.make_async_copy(
            rows_v.at[lax.rem(last - 1, 2)],
            out_dst(last - 1),
            wsem.at[lax.rem(last - 1, 2)],
        ).wait()


@jax.jit
def _emb(token_ids_t, weight):
    mesh = plsc.VectorSubcoreMesh(core_axis_name="c", subcore_axis_name="s")
    run = functools.partial(
        pl.kernel,
        mesh=mesh,
        out_type=jax.ShapeDtypeStruct((_H, _B, _D), jnp.float32),
        scratch_types=[
            pltpu.VMEM((2, _CHUNK), jnp.int32),
            pltpu.VMEM((2, _CHUNK, _D), jnp.float32),
            pltpu.SemaphoreType.DMA((2,)),
            pltpu.SemaphoreType.DMA((2,)),
        ],
        compiler_params=pltpu.CompilerParams(use_tc_tiling_on_sc=False),
    )(_emb_body)
    return run(token_ids_t, weight)


def kernel(token_ids, weight):
    table = jnp.pad(weight, ((0, 0), (0, 128 - _D))).reshape(-1, _D)
    idx4 = (token_ids.astype(jnp.int32) * 4).T
    out_t = _emb(idx4, table)
    return out_t.transpose(1, 0, 2)
